# baseline (speedup 1.0000x reference)
import functools

import numpy as np
import jax
import jax.numpy as jnp
from jax import lax
from jax.experimental import pallas as pl
from jax.experimental.pallas import tpu as pltpu
from jax.experimental.pallas import tpu_sc as plsc

N = 10000
D = 128
E = 320000
NC = 2
NS = 16
NW = NC * NS
NPAD = 10112
CHUNK = 128
NCHT = E // CHUNK
TRIPS = NCHT // NW
EXTRA = NCHT - TRIPS * NW

_MESH = dict(core_axis_name="c", subcore_axis_name="s")

_ZEROS1 = np.zeros((NPAD,), np.float32)
_ZEROS2 = np.zeros((NPAD, D), np.float32)
_ONES_C = np.ones((CHUNK,), np.float32)


def _acc_slice(t):
    return (t * 640, 512 if t == 15 else 640)


def _sc_degree(edge_index, zeros1, ones_c):
    @functools.partial(
        pl.kernel,
        mesh=plsc.VectorSubcoreMesh(**_MESH),
        out_type=jax.ShapeDtypeStruct((NC, NPAD), jnp.float32),
        scratch_types=[
            pltpu.VMEM((2, CHUNK), jnp.int32),
            pltpu.VMEM((2, CHUNK), jnp.int32),
            pltpu.VMEM((2, CHUNK), jnp.int32),
            pltpu.VMEM((CHUNK,), jnp.float32),
            pltpu.VMEM_SHARED((NPAD,), jnp.float32),
            pltpu.SemaphoreType.DMA,
            pltpu.SemaphoreType.DMA,
            pltpu.SemaphoreType.DMA,
        ],
    )
    def k(ei, z1, ones_hbm, out, idx0, idx1, idx2, ones_v, acc,
          ssb0, ssb1, ssb2):
        cid = lax.axis_index("c")
        sid = lax.axis_index("s")
        wid = cid * NS + sid
        idxv = (idx0, idx1, idx2)
        ssb = (ssb0, ssb1, ssb2)
        trips = TRIPS + jnp.where(wid < EXTRA, 1, 0)

        for t in range(NS):
            @pl.when(sid == t)
            def _(t=t):
                base_rows, nrows = _acc_slice(t)
                pltpu.sync_copy(z1.at[pl.ds(base_rows, nrows)],
                                acc.at[pl.ds(base_rows, nrows)])
        pltpu.sync_copy(ones_hbm, ones_v)
        plsc.subcore_barrier()

        def idx_load(j, b):
            base = pl.multiple_of((j * NW + wid) * CHUNK, CHUNK)
            pltpu.sync_copy(ei.at[:, pl.ds(base, CHUNK)], idxv[b])

        def sc(b):
            return pltpu.make_async_copy(ones_v, acc.at[idxv[b].at[1]], ssb[b])

        idx_load(0, 0)
        idx_load(1, 1)

        def step(j, b0, b2):
            pltpu.async_copy(ones_v, acc.at[idxv[b0].at[1]], ssb[b0],
                             add=True)
            @pl.when(j >= 1)
            def _():
                sc(b2).wait()
            @pl.when(j + 2 < trips)
            def _():
                idx_load(j + 2, b2)

        def body(j3, carry):
            step(j3 * 3, 0, 2)
            step(j3 * 3 + 1, 1, 0)
            step(j3 * 3 + 2, 2, 1)
            return carry

        lax.fori_loop(0, TRIPS // 3, body, 0)

        @pl.when(wid < EXTRA)
        def _():
            step(TRIPS, 0, 2)
            sc(0).wait()

        @pl.when(wid >= EXTRA)
        def _():
            sc(2).wait()

        plsc.subcore_barrier()
        for t in range(NS):
            @pl.when(sid == t)
            def _(t=t):
                base_rows, nrows = _acc_slice(t)
                pltpu.sync_copy(acc.at[pl.ds(base_rows, nrows)],
                                out.at[cid, pl.ds(base_rows, nrows)])

    return k(edge_index, zeros1, ones_c)


def _sc_aggregate(g, edge_index, zeros2):
    @functools.partial(
        pl.kernel,
        mesh=plsc.VectorSubcoreMesh(**_MESH),
        out_type=jax.ShapeDtypeStruct((NC, NPAD, D), jnp.float32),
        scratch_types=[
            pltpu.VMEM((2, CHUNK), jnp.int32),
            pltpu.VMEM((2, CHUNK), jnp.int32),
            pltpu.VMEM((2, CHUNK), jnp.int32),
            pltpu.VMEM((CHUNK, D), jnp.float32),
            pltpu.VMEM((CHUNK, D), jnp.float32),
            pltpu.VMEM((CHUNK, D), jnp.float32),
            pltpu.VMEM_SHARED((NPAD, D), jnp.float32),
            pltpu.SemaphoreType.DMA,
            pltpu.SemaphoreType.DMA,
            pltpu.SemaphoreType.DMA,
            pltpu.SemaphoreType.DMA,
        ],
    )
    def k(g_hbm, ei, z2, out, idx0, idx1, idx2, rows0, rows1, rows2, acc,
          gsem0, gsem1, gsem2, ssem):
        cid = lax.axis_index("c")
        sid = lax.axis_index("s")
        wid = cid * NS + sid
        idxv = (idx0, idx1, idx2)
        rows = (rows0, rows1, rows2)
        gsem = (gsem0, gsem1, gsem2)
        trips = TRIPS + jnp.where(wid < EXTRA, 1, 0)

        for t in range(NS):
            @pl.when(sid == t)
            def _(t=t):
                base_rows, nrows = _acc_slice(t)
                pltpu.sync_copy(z2.at[pl.ds(base_rows, nrows), :],
                                acc.at[pl.ds(base_rows, nrows), :])
        plsc.subcore_barrier()

        def idx_load(j, b):
            base = pl.multiple_of((j * NW + wid) * CHUNK, CHUNK)
            pltpu.sync_copy(ei.at[:, pl.ds(base, CHUNK)], idxv[b])

        def gather(b):
            return pltpu.make_async_copy(g_hbm.at[idxv[b].at[0]], rows[b],
                                         gsem[b])

        def scatter(b):
            return pltpu.make_async_copy(rows[b], acc.at[idxv[b].at[1]], ssem)

        idx_load(0, 0)
        gather(0).start()
        idx_load(1, 1)
        gather(1).start()

        def step(j, b0, b2):
            @pl.when(j >= 1)
            def _():
                scatter(b2).wait()
            @pl.when(j + 2 < trips)
            def _():
                idx_load(j + 2, b2)
                gather(b2).start()
            gather(b0).wait()
            pltpu.async_copy(rows[b0], acc.at[idxv[b0].at[1]], ssem,
                             add=True)

        def body(j3, carry):
            step(j3 * 3, 0, 2)
            step(j3 * 3 + 1, 1, 0)
            step(j3 * 3 + 2, 2, 1)
            return carry

        lax.fori_loop(0, TRIPS // 3, body, 0)

        @pl.when(wid < EXTRA)
        def _():
            step(TRIPS, 0, 2)
            scatter(0).wait()

        @pl.when(wid >= EXTRA)
        def _():
            scatter(2).wait()

        plsc.subcore_barrier()
        for t in range(NS):
            @pl.when(sid == t)
            def _(t=t):
                base_rows, nrows = _acc_slice(t)
                pltpu.sync_copy(acc.at[pl.ds(base_rows, nrows), :],
                                out.at[cid, pl.ds(base_rows, nrows), :])

    return k(g, edge_index, zeros2)


def _tc_scale(x, dis_col):
    def body(x_ref, d_ref, o_ref):
        o_ref[...] = x_ref[...] * d_ref[...]

    return pl.pallas_call(
        body, out_shape=jax.ShapeDtypeStruct((N, D), jnp.float32),
    )(x, dis_col)


def _tc_layer(P, g, dis_col, W, b, gamma, beta):
    def body(p_ref, g_ref, d_ref, w_ref, b_ref, ga_ref, be_ref, o_ref):
        s = p_ref[0, :N, :] + p_ref[1, :N, :] + g_ref[...]
        t = s * d_ref[...]
        u = lax.dot_general(t, w_ref[...], (((1,), (0,)), ((), ())),
                            preferred_element_type=jnp.float32,
                            precision=lax.Precision.HIGHEST) + b_ref[...]
        mean = jnp.mean(u, axis=0, keepdims=True)
        var = jnp.mean((u - mean) ** 2, axis=0, keepdims=True)
        v = (u - mean) * lax.rsqrt(var + 1e-5) * ga_ref[...] + be_ref[...]
        v = jnp.maximum(v, 0.0)
        o_ref[...] = v * d_ref[...]

    return pl.pallas_call(
        body, out_shape=jax.ShapeDtypeStruct((N, D), jnp.float32),
    )(P, g, dis_col, W, b, gamma, beta)


def _tc_final(P, g, dis_col, W, b, gamma, beta, lw1, lb1, lw2, lb2):
    def body(p_ref, g_ref, d_ref, w_ref, b_ref, ga_ref, be_ref,
             lw1_ref, lb1_ref, lw2_ref, lb2_ref, out_ref, pool_ref):
        s = p_ref[0, :N, :] + p_ref[1, :N, :] + g_ref[...]
        t = s * d_ref[...]
        u = lax.dot_general(t, w_ref[...], (((1,), (0,)), ((), ())),
                            preferred_element_type=jnp.float32,
                            precision=lax.Precision.HIGHEST) + b_ref[...]
        mean = jnp.mean(u, axis=0, keepdims=True)
        var = jnp.mean((u - mean) ** 2, axis=0, keepdims=True)
        h3 = (u - mean) * lax.rsqrt(var + 1e-5) * ga_ref[...] + be_ref[...]
        h3 = jnp.maximum(h3, 0.0)
        m1 = lax.dot_general(h3, lw1_ref[...], (((1,), (0,)), ((), ())),
                             preferred_element_type=jnp.float32,
                             precision=lax.Precision.HIGHEST) + lb1_ref[...]
        m1 = jnp.maximum(m1, 0.0)
        out_ref[...] = lax.dot_general(m1, lw2_ref[...], (((1,), (0,)), ((), ())),
                                       preferred_element_type=jnp.float32,
                                       precision=lax.Precision.HIGHEST) + lb2_ref[...]
        pool_ref[...] = jnp.mean(h3, axis=0, keepdims=True)

    return pl.pallas_call(
        body,
        out_shape=(jax.ShapeDtypeStruct((N, D), jnp.float32),
                   jax.ShapeDtypeStruct((1, D), jnp.float32)),
    )(P, g, dis_col, W, b, gamma, beta, lw1, lb1, lw2, lb2)


def kernel(x, edge_index, W0, b0, gamma0, beta0, W1, b1, gamma1, beta1,
           W2, b2, gamma2, beta2, lw1, lb1, lw2, lb2):
    zeros1 = _ZEROS1
    zeros2 = _ZEROS2
    ones_c = _ONES_C

    degp = _sc_degree(edge_index, zeros1, ones_c)
    deg = degp[0, :N] + degp[1, :N] + 1.0
    dis_col = lax.rsqrt(deg).reshape(N, 1)

    b0r, g0r, be0 = b0.reshape(1, D), gamma0.reshape(1, D), beta0.reshape(1, D)
    b1r, g1r, be1 = b1.reshape(1, D), gamma1.reshape(1, D), beta1.reshape(1, D)
    b2r, g2r, be2 = b2.reshape(1, D), gamma2.reshape(1, D), beta2.reshape(1, D)

    g = _tc_scale(x, dis_col)
    P = _sc_aggregate(g, edge_index, zeros2)
    g = _tc_layer(P, g, dis_col, W0, b0r, g0r, be0)
    P = _sc_aggregate(g, edge_index, zeros2)
    g = _tc_layer(P, g, dis_col, W1, b1r, g1r, be1)
    P = _sc_aggregate(g, edge_index, zeros2)
    out, pooled = _tc_final(P, g, dis_col, W2, b2r, g2r, be2,
                            lw1.reshape(D, D), lb1.reshape(1, D),
                            lw2.reshape(D, D), lb2.reshape(1, D))
    return (out, pooled)


# speedup vs baseline: 1.1596x; 1.1596x over previous
import functools

import numpy as np
import jax
import jax.numpy as jnp
from jax import lax
from jax.experimental import pallas as pl
from jax.experimental.pallas import tpu as pltpu
from jax.experimental.pallas import tpu_sc as plsc

N = 10000
D = 128
E = 320000
NC = 2
NS = 16
NW = NC * NS
NPAD = 10112
CHUNK = 128
NCHT = E // CHUNK
TRIPS = NCHT // NW
EXTRA = NCHT - TRIPS * NW

_MESH = dict(core_axis_name="c", subcore_axis_name="s")

_ZEROS1 = np.zeros((NPAD,), np.float32)
_ZEROS2 = np.zeros((NPAD, D), np.float32)
_ONES_C = np.ones((CHUNK,), np.float32)


def _acc_slice(t):
    return (t * 640, 512 if t == 15 else 640)


def _sc_degree(edge_index, zeros1, ones_c):
    @functools.partial(
        pl.kernel,
        mesh=plsc.VectorSubcoreMesh(**_MESH),
        out_type=jax.ShapeDtypeStruct((NC, NPAD), jnp.float32),
        scratch_types=[
            pltpu.VMEM((2, CHUNK), jnp.int32),
            pltpu.VMEM((2, CHUNK), jnp.int32),
            pltpu.VMEM((2, CHUNK), jnp.int32),
            pltpu.VMEM((CHUNK,), jnp.float32),
            pltpu.VMEM_SHARED((NPAD,), jnp.float32),
            pltpu.SemaphoreType.DMA,
            pltpu.SemaphoreType.DMA,
            pltpu.SemaphoreType.DMA,
        ],
    )
    def k(ei, z1, ones_hbm, out, idx0, idx1, idx2, ones_v, acc,
          ssb0, ssb1, ssb2):
        cid = lax.axis_index("c")
        sid = lax.axis_index("s")
        wid = cid * NS + sid
        idxv = (idx0, idx1, idx2)
        ssb = (ssb0, ssb1, ssb2)
        trips = TRIPS + jnp.where(wid < EXTRA, 1, 0)

        for t in range(NS):
            @pl.when(sid == t)
            def _(t=t):
                base_rows, nrows = _acc_slice(t)
                pltpu.sync_copy(z1.at[pl.ds(base_rows, nrows)],
                                acc.at[pl.ds(base_rows, nrows)])
        pltpu.sync_copy(ones_hbm, ones_v)
        plsc.subcore_barrier()

        def idx_load(j, b):
            base = pl.multiple_of((j * NW + wid) * CHUNK, CHUNK)
            pltpu.sync_copy(ei.at[:, pl.ds(base, CHUNK)], idxv[b])

        def sc(b):
            return pltpu.make_async_copy(ones_v, acc.at[idxv[b].at[1]], ssb[b])

        idx_load(0, 0)
        idx_load(1, 1)

        def step(j, b0, b2):
            pltpu.async_copy(ones_v, acc.at[idxv[b0].at[1]], ssb[b0],
                             add=True)
            @pl.when(j >= 1)
            def _():
                sc(b2).wait()
            @pl.when(j + 2 < trips)
            def _():
                idx_load(j + 2, b2)

        def body(j3, carry):
            step(j3 * 3, 0, 2)
            step(j3 * 3 + 1, 1, 0)
            step(j3 * 3 + 2, 2, 1)
            return carry

        lax.fori_loop(0, TRIPS // 3, body, 0)

        @pl.when(wid < EXTRA)
        def _():
            step(TRIPS, 0, 2)
            sc(0).wait()

        @pl.when(wid >= EXTRA)
        def _():
            sc(2).wait()

        plsc.subcore_barrier()
        for t in range(NS):
            @pl.when(sid == t)
            def _(t=t):
                base_rows, nrows = _acc_slice(t)
                pltpu.sync_copy(acc.at[pl.ds(base_rows, nrows)],
                                out.at[cid, pl.ds(base_rows, nrows)])

    return k(edge_index, zeros1, ones_c)


def _sc_aggregate(g, edge_index, zeros2):
    @functools.partial(
        pl.kernel,
        mesh=plsc.VectorSubcoreMesh(**_MESH),
        out_type=jax.ShapeDtypeStruct((NC, NPAD, D), jnp.float32),
        scratch_types=[
            pltpu.VMEM((2, CHUNK), jnp.int32),
            pltpu.VMEM((2, CHUNK), jnp.int32),
            pltpu.VMEM((2, CHUNK), jnp.int32),
            pltpu.VMEM((2, CHUNK), jnp.int32),
            pltpu.VMEM((CHUNK, D), jnp.float32),
            pltpu.VMEM((CHUNK, D), jnp.float32),
            pltpu.VMEM((CHUNK, D), jnp.float32),
            pltpu.VMEM_SHARED((NPAD, D), jnp.float32),
            pltpu.SemaphoreType.DMA,
            pltpu.SemaphoreType.DMA,
            pltpu.SemaphoreType.DMA,
            pltpu.SemaphoreType.DMA,
            pltpu.SemaphoreType.DMA,
            pltpu.SemaphoreType.DMA,
            pltpu.SemaphoreType.DMA,
            pltpu.SemaphoreType.DMA,
        ],
    )
    def k(g_hbm, ei, z2, out, idx0, idx1, idx2, idx3,
          rows0, rows1, rows2, acc,
          isem0, isem1, isem2, isem3, gsem0, gsem1, gsem2, ssem):
        cid = lax.axis_index("c")
        sid = lax.axis_index("s")
        wid = cid * NS + sid
        idxv = (idx0, idx1, idx2, idx3)
        isem = (isem0, isem1, isem2, isem3)
        rows = (rows0, rows1, rows2)
        gsem = (gsem0, gsem1, gsem2)
        trips = TRIPS + jnp.where(wid < EXTRA, 1, 0)

        for t in range(NS):
            @pl.when(sid == t)
            def _(t=t):
                base_rows, nrows = _acc_slice(t)
                pltpu.sync_copy(z2.at[pl.ds(base_rows, nrows), :],
                                acc.at[pl.ds(base_rows, nrows), :])
        plsc.subcore_barrier()

        def idx_copy(j, bi):
            base = pl.multiple_of((j * NW + wid) * CHUNK, CHUNK)
            return pltpu.make_async_copy(ei.at[:, pl.ds(base, CHUNK)],
                                         idxv[bi], isem[bi])

        def gather(br, bi):
            return pltpu.make_async_copy(g_hbm.at[idxv[bi].at[0]], rows[br],
                                         gsem[br])

        def scatter(br, bi):
            return pltpu.make_async_copy(rows[br], acc.at[idxv[bi].at[1]],
                                         ssem)

        c = idx_copy(0, 0); c.start(); c.wait()
        gather(0, 0).start()
        c = idx_copy(1, 1); c.start(); c.wait()
        gather(1, 1).start()
        idx_copy(2, 2).start()

        def step(j, u):
            br0, bi0 = u % 3, u % 4
            br2, bi2 = (u + 2) % 3, (u + 2) % 4
            bi3 = (u + 3) % 4
            @pl.when(j >= 1)
            def _():
                scatter(br2, bi3).wait()
            @pl.when(j + 3 < trips)
            def _():
                idx_copy(j + 3, bi3).start()
            @pl.when(j + 2 < trips)
            def _():
                idx_copy(j + 2, bi2).wait()
                gather(br2, bi2).start()
            gather(br0, bi0).wait()
            pltpu.async_copy(rows[br0], acc.at[idxv[bi0].at[1]], ssem,
                             add=True)

        def body(j12, carry):
            for u in range(12):
                step(j12 * 12 + u, u)
            return carry

        lax.fori_loop(0, TRIPS // 12, body, 0)
        for j in range(TRIPS - TRIPS % 12, TRIPS):
            step(j, j % 12)

        @pl.when(wid < EXTRA)
        def _():
            step(TRIPS, TRIPS % 12)
            scatter(TRIPS % 3, TRIPS % 4).wait()

        @pl.when(wid >= EXTRA)
        def _():
            scatter((TRIPS - 1) % 3, (TRIPS - 1) % 4).wait()

        plsc.subcore_barrier()
        for t in range(NS):
            @pl.when(sid == t)
            def _(t=t):
                base_rows, nrows = _acc_slice(t)
                pltpu.sync_copy(acc.at[pl.ds(base_rows, nrows), :],
                                out.at[cid, pl.ds(base_rows, nrows), :])

    return k(g, edge_index, zeros2)


def _tc_scale(x, dis_col):
    def body(x_ref, d_ref, o_ref):
        o_ref[...] = x_ref[...] * d_ref[...]

    return pl.pallas_call(
        body, out_shape=jax.ShapeDtypeStruct((N, D), jnp.float32),
    )(x, dis_col)


def _tc_layer(P, g, dis_col, W, b, gamma, beta):
    def body(p_ref, g_ref, d_ref, w_ref, b_ref, ga_ref, be_ref, o_ref):
        s = p_ref[0, :N, :] + p_ref[1, :N, :] + g_ref[...]
        t = s * d_ref[...]
        u = lax.dot_general(t, w_ref[...], (((1,), (0,)), ((), ())),
                            preferred_element_type=jnp.float32,
                            precision=lax.Precision.HIGHEST) + b_ref[...]
        mean = jnp.mean(u, axis=0, keepdims=True)
        var = jnp.mean((u - mean) ** 2, axis=0, keepdims=True)
        v = (u - mean) * lax.rsqrt(var + 1e-5) * ga_ref[...] + be_ref[...]
        v = jnp.maximum(v, 0.0)
        o_ref[...] = v * d_ref[...]

    return pl.pallas_call(
        body, out_shape=jax.ShapeDtypeStruct((N, D), jnp.float32),
    )(P, g, dis_col, W, b, gamma, beta)


def _tc_final(P, g, dis_col, W, b, gamma, beta, lw1, lb1, lw2, lb2):
    def body(p_ref, g_ref, d_ref, w_ref, b_ref, ga_ref, be_ref,
             lw1_ref, lb1_ref, lw2_ref, lb2_ref, out_ref, pool_ref):
        s = p_ref[0, :N, :] + p_ref[1, :N, :] + g_ref[...]
        t = s * d_ref[...]
        u = lax.dot_general(t, w_ref[...], (((1,), (0,)), ((), ())),
                            preferred_element_type=jnp.float32,
                            precision=lax.Precision.HIGHEST) + b_ref[...]
        mean = jnp.mean(u, axis=0, keepdims=True)
        var = jnp.mean((u - mean) ** 2, axis=0, keepdims=True)
        h3 = (u - mean) * lax.rsqrt(var + 1e-5) * ga_ref[...] + be_ref[...]
        h3 = jnp.maximum(h3, 0.0)
        m1 = lax.dot_general(h3, lw1_ref[...], (((1,), (0,)), ((), ())),
                             preferred_element_type=jnp.float32,
                             precision=lax.Precision.HIGHEST) + lb1_ref[...]
        m1 = jnp.maximum(m1, 0.0)
        out_ref[...] = lax.dot_general(m1, lw2_ref[...], (((1,), (0,)), ((), ())),
                                       preferred_element_type=jnp.float32,
                                       precision=lax.Precision.HIGHEST) + lb2_ref[...]
        pool_ref[...] = jnp.mean(h3, axis=0, keepdims=True)

    return pl.pallas_call(
        body,
        out_shape=(jax.ShapeDtypeStruct((N, D), jnp.float32),
                   jax.ShapeDtypeStruct((1, D), jnp.float32)),
    )(P, g, dis_col, W, b, gamma, beta, lw1, lb1, lw2, lb2)


def kernel(x, edge_index, W0, b0, gamma0, beta0, W1, b1, gamma1, beta1,
           W2, b2, gamma2, beta2, lw1, lb1, lw2, lb2):
    zeros1 = _ZEROS1
    zeros2 = _ZEROS2
    ones_c = _ONES_C

    degp = _sc_degree(edge_index, zeros1, ones_c)
    deg = degp[0, :N] + degp[1, :N] + 1.0
    dis_col = lax.rsqrt(deg).reshape(N, 1)

    b0r, g0r, be0 = b0.reshape(1, D), gamma0.reshape(1, D), beta0.reshape(1, D)
    b1r, g1r, be1 = b1.reshape(1, D), gamma1.reshape(1, D), beta1.reshape(1, D)
    b2r, g2r, be2 = b2.reshape(1, D), gamma2.reshape(1, D), beta2.reshape(1, D)

    g = _tc_scale(x, dis_col)
    P = _sc_aggregate(g, edge_index, zeros2)
    g = _tc_layer(P, g, dis_col, W0, b0r, g0r, be0)
    P = _sc_aggregate(g, edge_index, zeros2)
    g = _tc_layer(P, g, dis_col, W1, b1r, g1r, be1)
    P = _sc_aggregate(g, edge_index, zeros2)
    out, pooled = _tc_final(P, g, dis_col, W2, b2r, g2r, be2,
                            lw1.reshape(D, D), lb1.reshape(1, D),
                            lw2.reshape(D, D), lb2.reshape(1, D))
    return (out, pooled)


# speedup vs baseline: 1.2163x; 1.0489x over previous
import functools

import numpy as np
import jax
import jax.numpy as jnp
from jax import lax
from jax.experimental import pallas as pl
from jax.experimental.pallas import tpu as pltpu
from jax.experimental.pallas import tpu_sc as plsc

N = 10000
D = 128
E = 320000
NC = 2
NS = 16
NW = NC * NS
NPAD = 10112
CHUNK = 128
NCHT = E // CHUNK
TRIPS = NCHT // NW
EXTRA = NCHT - TRIPS * NW

_MESH = dict(core_axis_name="c", subcore_axis_name="s")

_ZEROS1 = np.zeros((NPAD,), np.float32)
_ZEROS2 = np.zeros((NPAD, D), np.float32)
_ONES_C = np.ones((CHUNK,), np.float32)


def _acc_slice(t):
    return (t * 640, 512 if t == 15 else 640)


def _sc_degree(edge_index, zeros1, ones_c):
    @functools.partial(
        pl.kernel,
        mesh=plsc.VectorSubcoreMesh(**_MESH),
        out_type=jax.ShapeDtypeStruct((NC, NPAD), jnp.float32),
        scratch_types=[
            pltpu.VMEM((2, CHUNK), jnp.int32),
            pltpu.VMEM((2, CHUNK), jnp.int32),
            pltpu.VMEM((2, CHUNK), jnp.int32),
            pltpu.VMEM((2, CHUNK), jnp.int32),
            pltpu.VMEM((CHUNK,), jnp.float32),
            pltpu.VMEM_SHARED((NPAD,), jnp.float32),
            pltpu.SemaphoreType.DMA,
            pltpu.SemaphoreType.DMA,
            pltpu.SemaphoreType.DMA,
            pltpu.SemaphoreType.DMA,
            pltpu.SemaphoreType.DMA,
            pltpu.SemaphoreType.DMA,
            pltpu.SemaphoreType.DMA,
            pltpu.SemaphoreType.DMA,
        ],
    )
    def k(ei, z1, ones_hbm, out, idx0, idx1, idx2, idx3, ones_v, acc,
          isem0, isem1, isem2, isem3, ssb0, ssb1, ssb2, ssb3):
        cid = lax.axis_index("c")
        sid = lax.axis_index("s")
        wid = cid * NS + sid
        idxv = (idx0, idx1, idx2, idx3)
        isem = (isem0, isem1, isem2, isem3)
        ssb = (ssb0, ssb1, ssb2, ssb3)
        trips = TRIPS + jnp.where(wid < EXTRA, 1, 0)

        for t in range(NS):
            @pl.when(sid == t)
            def _(t=t):
                base_rows, nrows = _acc_slice(t)
                pltpu.sync_copy(z1.at[pl.ds(base_rows, nrows)],
                                acc.at[pl.ds(base_rows, nrows)])
        pltpu.sync_copy(ones_hbm, ones_v)
        plsc.subcore_barrier()

        def idx_copy(j, bi):
            base = pl.multiple_of((j * NW + wid) * CHUNK, CHUNK)
            return pltpu.make_async_copy(ei.at[:, pl.ds(base, CHUNK)],
                                         idxv[bi], isem[bi])

        def sc(bi):
            return pltpu.make_async_copy(ones_v, acc.at[idxv[bi].at[1]],
                                         ssb[bi])

        c = idx_copy(0, 0); c.start(); c.wait()
        c = idx_copy(1, 1); c.start(); c.wait()
        idx_copy(2, 2).start()

        def step(j, u):
            bi0, bi2, bi3 = u % 4, (u + 2) % 4, (u + 3) % 4
            pltpu.async_copy(ones_v, acc.at[idxv[bi0].at[1]], ssb[bi0],
                             add=True)
            @pl.when(j >= 1)
            def _():
                sc(bi3).wait()
            @pl.when(j + 3 < trips)
            def _():
                idx_copy(j + 3, bi3).start()
            @pl.when(j + 2 < trips)
            def _():
                idx_copy(j + 2, bi2).wait()

        def body(j4, carry):
            for u in range(4):
                step(j4 * 4 + u, u)
            return carry

        lax.fori_loop(0, TRIPS // 4, body, 0)
        for j in range(TRIPS - TRIPS % 4, TRIPS):
            step(j, j % 4)

        @pl.when(wid < EXTRA)
        def _():
            step(TRIPS, TRIPS % 4)
            sc(TRIPS % 4).wait()

        @pl.when(wid >= EXTRA)
        def _():
            sc((TRIPS - 1) % 4).wait()

        plsc.subcore_barrier()
        for t in range(NS):
            @pl.when(sid == t)
            def _(t=t):
                base_rows, nrows = _acc_slice(t)
                pltpu.sync_copy(acc.at[pl.ds(base_rows, nrows)],
                                out.at[cid, pl.ds(base_rows, nrows)])

    return k(edge_index, zeros1, ones_c)


def _sc_aggregate(g, edge_index, zeros2):
    @functools.partial(
        pl.kernel,
        mesh=plsc.VectorSubcoreMesh(**_MESH),
        out_type=jax.ShapeDtypeStruct((NC, NPAD, D), jnp.float32),
        scratch_types=[
            pltpu.VMEM((2, CHUNK), jnp.int32),
            pltpu.VMEM((2, CHUNK), jnp.int32),
            pltpu.VMEM((2, CHUNK), jnp.int32),
            pltpu.VMEM((2, CHUNK), jnp.int32),
            pltpu.VMEM((CHUNK, D), jnp.float32),
            pltpu.VMEM((CHUNK, D), jnp.float32),
            pltpu.VMEM((CHUNK, D), jnp.float32),
            pltpu.VMEM_SHARED((NPAD, D), jnp.float32),
            pltpu.SemaphoreType.DMA,
            pltpu.SemaphoreType.DMA,
            pltpu.SemaphoreType.DMA,
            pltpu.SemaphoreType.DMA,
            pltpu.SemaphoreType.DMA,
            pltpu.SemaphoreType.DMA,
            pltpu.SemaphoreType.DMA,
            pltpu.SemaphoreType.DMA,
        ],
    )
    def k(g_hbm, ei, z2, out, idx0, idx1, idx2, idx3,
          rows0, rows1, rows2, acc,
          isem0, isem1, isem2, isem3, gsem0, gsem1, gsem2, ssem):
        cid = lax.axis_index("c")
        sid = lax.axis_index("s")
        wid = cid * NS + sid
        idxv = (idx0, idx1, idx2, idx3)
        isem = (isem0, isem1, isem2, isem3)
        rows = (rows0, rows1, rows2)
        gsem = (gsem0, gsem1, gsem2)
        trips = TRIPS + jnp.where(wid < EXTRA, 1, 0)

        for t in range(NS):
            @pl.when(sid == t)
            def _(t=t):
                base_rows, nrows = _acc_slice(t)
                pltpu.sync_copy(z2.at[pl.ds(base_rows, nrows), :],
                                acc.at[pl.ds(base_rows, nrows), :])
        plsc.subcore_barrier()

        def idx_copy(j, bi):
            base = pl.multiple_of((j * NW + wid) * CHUNK, CHUNK)
            return pltpu.make_async_copy(ei.at[:, pl.ds(base, CHUNK)],
                                         idxv[bi], isem[bi])

        def gather(br, bi):
            return pltpu.make_async_copy(g_hbm.at[idxv[bi].at[0]], rows[br],
                                         gsem[br])

        def scatter(br, bi):
            return pltpu.make_async_copy(rows[br], acc.at[idxv[bi].at[1]],
                                         ssem)

        c = idx_copy(0, 0); c.start(); c.wait()
        gather(0, 0).start()
        c = idx_copy(1, 1); c.start(); c.wait()
        gather(1, 1).start()
        idx_copy(2, 2).start()

        def step(j, u):
            br0, bi0 = u % 3, u % 4
            br2, bi2 = (u + 2) % 3, (u + 2) % 4
            bi3 = (u + 3) % 4
            @pl.when(j >= 1)
            def _():
                scatter(br2, bi3).wait()
            @pl.when(j + 3 < trips)
            def _():
                idx_copy(j + 3, bi3).start()
            @pl.when(j + 2 < trips)
            def _():
                idx_copy(j + 2, bi2).wait()
                gather(br2, bi2).start()
            gather(br0, bi0).wait()
            pltpu.async_copy(rows[br0], acc.at[idxv[bi0].at[1]], ssem,
                             add=True)

        def body(j12, carry):
            for u in range(12):
                step(j12 * 12 + u, u)
            return carry

        lax.fori_loop(0, TRIPS // 12, body, 0)
        for j in range(TRIPS - TRIPS % 12, TRIPS):
            step(j, j % 12)

        @pl.when(wid < EXTRA)
        def _():
            step(TRIPS, TRIPS % 12)
            scatter(TRIPS % 3, TRIPS % 4).wait()

        @pl.when(wid >= EXTRA)
        def _():
            scatter((TRIPS - 1) % 3, (TRIPS - 1) % 4).wait()

        plsc.subcore_barrier()
        for t in range(NS):
            @pl.when(sid == t)
            def _(t=t):
                base_rows, nrows = _acc_slice(t)
                pltpu.sync_copy(acc.at[pl.ds(base_rows, nrows), :],
                                out.at[cid, pl.ds(base_rows, nrows), :])

    return k(g, edge_index, zeros2)


def _tc_scale(x, dis_col):
    def body(x_ref, d_ref, o_ref):
        o_ref[...] = x_ref[...] * d_ref[...]

    return pl.pallas_call(
        body, out_shape=jax.ShapeDtypeStruct((N, D), jnp.float32),
    )(x, dis_col)


def _tc_layer(P, g, dis_col, W, b, gamma, beta):
    def body(p_ref, g_ref, d_ref, w_ref, b_ref, ga_ref, be_ref, o_ref):
        s = p_ref[0, :N, :] + p_ref[1, :N, :] + g_ref[...]
        t = s * d_ref[...]
        u = lax.dot_general(t, w_ref[...], (((1,), (0,)), ((), ())),
                            preferred_element_type=jnp.float32,
                            precision=lax.Precision.HIGHEST) + b_ref[...]
        mean = jnp.mean(u, axis=0, keepdims=True)
        var = jnp.mean((u - mean) ** 2, axis=0, keepdims=True)
        v = (u - mean) * lax.rsqrt(var + 1e-5) * ga_ref[...] + be_ref[...]
        v = jnp.maximum(v, 0.0)
        o_ref[...] = v * d_ref[...]

    return pl.pallas_call(
        body, out_shape=jax.ShapeDtypeStruct((N, D), jnp.float32),
    )(P, g, dis_col, W, b, gamma, beta)


def _tc_final(P, g, dis_col, W, b, gamma, beta, lw1, lb1, lw2, lb2):
    def body(p_ref, g_ref, d_ref, w_ref, b_ref, ga_ref, be_ref,
             lw1_ref, lb1_ref, lw2_ref, lb2_ref, out_ref, pool_ref):
        s = p_ref[0, :N, :] + p_ref[1, :N, :] + g_ref[...]
        t = s * d_ref[...]
        u = lax.dot_general(t, w_ref[...], (((1,), (0,)), ((), ())),
                            preferred_element_type=jnp.float32,
                            precision=lax.Precision.HIGHEST) + b_ref[...]
        mean = jnp.mean(u, axis=0, keepdims=True)
        var = jnp.mean((u - mean) ** 2, axis=0, keepdims=True)
        h3 = (u - mean) * lax.rsqrt(var + 1e-5) * ga_ref[...] + be_ref[...]
        h3 = jnp.maximum(h3, 0.0)
        m1 = lax.dot_general(h3, lw1_ref[...], (((1,), (0,)), ((), ())),
                             preferred_element_type=jnp.float32,
                             precision=lax.Precision.HIGHEST) + lb1_ref[...]
        m1 = jnp.maximum(m1, 0.0)
        out_ref[...] = lax.dot_general(m1, lw2_ref[...], (((1,), (0,)), ((), ())),
                                       preferred_element_type=jnp.float32,
                                       precision=lax.Precision.HIGHEST) + lb2_ref[...]
        pool_ref[...] = jnp.mean(h3, axis=0, keepdims=True)

    return pl.pallas_call(
        body,
        out_shape=(jax.ShapeDtypeStruct((N, D), jnp.float32),
                   jax.ShapeDtypeStruct((1, D), jnp.float32)),
    )(P, g, dis_col, W, b, gamma, beta, lw1, lb1, lw2, lb2)


def kernel(x, edge_index, W0, b0, gamma0, beta0, W1, b1, gamma1, beta1,
           W2, b2, gamma2, beta2, lw1, lb1, lw2, lb2):
    zeros1 = _ZEROS1
    zeros2 = _ZEROS2
    ones_c = _ONES_C

    degp = _sc_degree(edge_index, zeros1, ones_c)
    deg = degp[0, :N] + degp[1, :N] + 1.0
    dis_col = lax.rsqrt(deg).reshape(N, 1)

    b0r, g0r, be0 = b0.reshape(1, D), gamma0.reshape(1, D), beta0.reshape(1, D)
    b1r, g1r, be1 = b1.reshape(1, D), gamma1.reshape(1, D), beta1.reshape(1, D)
    b2r, g2r, be2 = b2.reshape(1, D), gamma2.reshape(1, D), beta2.reshape(1, D)

    g = _tc_scale(x, dis_col)
    P = _sc_aggregate(g, edge_index, zeros2)
    g = _tc_layer(P, g, dis_col, W0, b0r, g0r, be0)
    P = _sc_aggregate(g, edge_index, zeros2)
    g = _tc_layer(P, g, dis_col, W1, b1r, g1r, be1)
    P = _sc_aggregate(g, edge_index, zeros2)
    out, pooled = _tc_final(P, g, dis_col, W2, b2r, g2r, be2,
                            lw1.reshape(D, D), lb1.reshape(1, D),
                            lw2.reshape(D, D), lb2.reshape(1, D))
    return (out, pooled)
